# baseline (speedup 1.0000x reference)
import functools

import jax
import jax.numpy as jnp
from jax import lax
from jax.experimental import pallas as pl
from jax.experimental.pallas import tpu as pltpu
from jax.experimental.pallas import tpu_sc as plsc


def _pooled_sum_sc(tokens_flat, embed_table, B, T, D):
    info = plsc.get_sparse_core_info()
    NC, NS, L = info.num_cores, info.num_subcores, info.num_lanes
    NW = NC * NS
    assert B % NW == 0
    b_per_w = B // NW
    nvec = D // L

    mesh = plsc.VectorSubcoreMesh(core_axis_name="c", subcore_axis_name="s")

    @functools.partial(
        pl.kernel,
        mesh=mesh,
        compiler_params=pltpu.CompilerParams(use_tc_tiling_on_sc=False),
        out_type=jax.ShapeDtypeStruct((B, D), jnp.float32),
        scratch_types=[
            pltpu.VMEM((b_per_w * T,), jnp.int32),
            pltpu.VMEM((T, D), jnp.float32),
            pltpu.VMEM((b_per_w, D), jnp.float32),
            pltpu.SemaphoreType.DMA,
        ],
    )
    def k(tokens_hbm, table_hbm, out_hbm, idx_v, rows_v, acc_v, sem):
        wid = lax.axis_index("s") * NC + lax.axis_index("c")
        base = wid * b_per_w
        pltpu.sync_copy(tokens_hbm.at[pl.ds(base * T, b_per_w * T)], idx_v)

        def row_body(r, _):
            pltpu.async_copy(
                table_hbm.at[idx_v.at[pl.ds(r * T, T)]], rows_v, sem
            ).wait()

            def tok_body(t, acc):
                return tuple(
                    acc[i] + rows_v[t, pl.ds(i * L, L)] for i in range(nvec)
                )

            zeros = tuple(jnp.zeros((L,), jnp.float32) for _ in range(nvec))
            acc = lax.fori_loop(0, T, tok_body, zeros, unroll=4)
            for i in range(nvec):
                acc_v[r, pl.ds(i * L, L)] = acc[i]
            return 0

        lax.fori_loop(0, b_per_w, row_body, 0)
        pltpu.sync_copy(acc_v, out_hbm.at[pl.ds(base, b_per_w)])

    return k(tokens_flat, embed_table)


def _mlp_tc(pooled_sum, W1, b1, W2, b2, T):
    B, D = pooled_sum.shape

    def body(x_ref, w1_ref, b1_ref, w2_ref, b2_ref, o_ref):
        x = x_ref[...] * (1.0 / T)
        h = jnp.dot(x, w1_ref[...], preferred_element_type=jnp.float32)
        h = jnp.maximum(h + b1_ref[...], 0.0)
        o_ref[...] = (
            jnp.dot(h, w2_ref[...], preferred_element_type=jnp.float32)
            + b2_ref[...]
        )

    out = pl.pallas_call(
        body,
        out_shape=jax.ShapeDtypeStruct((B, 1), jnp.float32),
    )(pooled_sum, W1, b1.reshape(1, D), W2, b2.reshape(1, 1))
    return jnp.squeeze(out, axis=-1)


def kernel(tokens, embed_table, W1, b1, W2, b2):
    B, T = tokens.shape
    V, D = embed_table.shape
    tokens_flat = tokens.reshape(B * T)
    pooled_sum = _pooled_sum_sc(tokens_flat, embed_table, B, T, D)
    return _mlp_tc(pooled_sum, W1, b1, W2, b2, T)


# speedup vs baseline: 1.1945x; 1.1945x over previous
import functools

import jax
import jax.numpy as jnp
from jax import lax
from jax.experimental import pallas as pl
from jax.experimental.pallas import tpu as pltpu
from jax.experimental.pallas import tpu_sc as plsc


def _pooled_sum_sc(tokens_flat, embed_table, B, T, D):
    info = plsc.get_sparse_core_info()
    NC, NS, L = info.num_cores, info.num_subcores, info.num_lanes
    NW = NC * NS
    assert B % NW == 0
    b_per_w = B // NW
    nvec = D // L

    mesh = plsc.VectorSubcoreMesh(core_axis_name="c", subcore_axis_name="s")

    NBUF = 4

    @functools.partial(
        pl.kernel,
        mesh=mesh,
        compiler_params=pltpu.CompilerParams(use_tc_tiling_on_sc=False),
        out_type=jax.ShapeDtypeStruct((B, D), jnp.float32),
        scratch_types=[
            pltpu.VMEM((b_per_w * T,), jnp.int32),
            pltpu.VMEM((NBUF, T, D), jnp.float32),
            pltpu.VMEM((b_per_w, D), jnp.float32),
            pltpu.SemaphoreType.DMA((NBUF,)),
        ],
    )
    def k(tokens_hbm, table_hbm, out_hbm, idx_v, rows_v, acc_v, sems):
        wid = lax.axis_index("s") * NC + lax.axis_index("c")
        base = wid * b_per_w
        pltpu.sync_copy(tokens_hbm.at[pl.ds(base * T, b_per_w * T)], idx_v)

        def gather_start(row, b):
            pltpu.async_copy(
                table_hbm.at[idx_v.at[pl.ds(row * T, T)]],
                rows_v.at[b],
                sems.at[b],
            )

        def gather_wait(row, b):
            pltpu.make_async_copy(
                table_hbm.at[idx_v.at[pl.ds(row * T, T)]],
                rows_v.at[b],
                sems.at[b],
            ).wait()

        for b in range(NBUF):
            gather_start(b, b)

        def group_body(i, _):
            r0 = i * NBUF
            for b in range(NBUF):
                r = r0 + b
                gather_wait(r, b)

                def tok_body(t, acc):
                    return tuple(
                        acc[j] + rows_v[b, t, pl.ds(j * L, L)]
                        for j in range(nvec)
                    )

                zeros = tuple(jnp.zeros((L,), jnp.float32) for _ in range(nvec))
                acc = lax.fori_loop(0, T, tok_body, zeros, unroll=4)
                for j in range(nvec):
                    acc_v[r, pl.ds(j * L, L)] = acc[j]

                @pl.when(r + NBUF < b_per_w)
                def _():
                    gather_start(r + NBUF, b)

            return 0

        lax.fori_loop(0, b_per_w // NBUF, group_body, 0)
        pltpu.sync_copy(acc_v, out_hbm.at[pl.ds(base, b_per_w)])

    return k(tokens_flat, embed_table)


def _mlp_tc(pooled_sum, W1, b1, W2, b2, T):
    B, D = pooled_sum.shape

    def body(x_ref, w1_ref, b1_ref, w2_ref, b2_ref, o_ref):
        x = x_ref[...] * (1.0 / T)
        h = jnp.dot(x, w1_ref[...], preferred_element_type=jnp.float32)
        h = jnp.maximum(h + b1_ref[...], 0.0)
        o_ref[...] = (
            jnp.dot(h, w2_ref[...], preferred_element_type=jnp.float32)
            + b2_ref[...]
        )

    out = pl.pallas_call(
        body,
        out_shape=jax.ShapeDtypeStruct((B, 1), jnp.float32),
    )(pooled_sum, W1, b1.reshape(1, D), W2, b2.reshape(1, 1))
    return jnp.squeeze(out, axis=-1)


def kernel(tokens, embed_table, W1, b1, W2, b2):
    B, T = tokens.shape
    V, D = embed_table.shape
    tokens_flat = tokens.reshape(B * T)
    pooled_sum = _pooled_sum_sc(tokens_flat, embed_table, B, T, D)
    return _mlp_tc(pooled_sum, W1, b1, W2, b2, T)
